# speedup vs baseline: 1.3503x; 1.3503x over previous
import functools
import math

import jax
import jax.numpy as jnp
from jax import lax
from jax.experimental import pallas as pl
from jax.experimental.pallas import tpu as pltpu
from jax.experimental.pallas import tpu_sc as plsc

B, N, DEG, D, H = 2, 10000, 16, 128, 4
HD = D // H
NDEG = N * DEG
NW = 32
ROWS_PER_W = 5120
NPAD = NW * ROWS_PER_W
CH = 128
NCH = ROWS_PER_W // CH
TW = 4 * D

NB = 200
GB = NB * DEG
NBLK = N // NB

NBP = 2000
NPRE = N // NBP


def _pre_body(x_ref, wkT_ref, bk_ref, wvT_ref, bv_ref, out_ref):
    wkT = wkT_ref[...]
    wvT = wvT_ref[...]
    for b in range(B):
        xb = x_ref[b]
        out_ref[:, b * D:(b + 1) * D] = (
            jnp.dot(xb, wkT, preferred_element_type=jnp.float32) + bk_ref[...])
        out_ref[:, 2 * D + b * D:2 * D + (b + 1) * D] = (
            jnp.dot(xb, wvT, preferred_element_type=jnp.float32) + bv_ref[...])


def _build_table(x, WkT, bk, WvT, bv):
    return pl.pallas_call(
        _pre_body,
        grid=(NPRE,),
        in_specs=[
            pl.BlockSpec((B, NBP, D), lambda i: (0, i, 0)),
            pl.BlockSpec((D, D), lambda i: (0, 0)),
            pl.BlockSpec((1, D), lambda i: (0, 0)),
            pl.BlockSpec((D, D), lambda i: (0, 0)),
            pl.BlockSpec((1, D), lambda i: (0, 0)),
        ],
        out_specs=pl.BlockSpec((NBP, TW), lambda i: (i, 0)),
        out_shape=jax.ShapeDtypeStruct((N, TW), jnp.float32),
    )(x, WkT, bk, WvT, bv)


def _gather_body(table_hbm, idx_hbm, out_hbm, idx_v, rows_v, sem):
    c = lax.axis_index("c")
    s = lax.axis_index("s")
    wid = s * 2 + c
    base = wid * ROWS_PER_W
    for j in range(NCH):
        off = base + j * CH
        pltpu.sync_copy(idx_hbm.at[pl.ds(off, CH)], idx_v)
        pltpu.async_copy(table_hbm.at[idx_v], rows_v, sem).wait()
        pltpu.sync_copy(rows_v, out_hbm.at[pl.ds(off, CH)])


def _gather_rows(table, idx_pad):
    mesh = plsc.VectorSubcoreMesh(core_axis_name="c", subcore_axis_name="s")
    k = pl.kernel(
        _gather_body,
        out_type=jax.ShapeDtypeStruct((NPAD, TW), jnp.float32),
        mesh=mesh,
        scratch_types=[
            pltpu.VMEM((CH,), jnp.int32),
            pltpu.VMEM((CH, TW), jnp.float32),
            pltpu.SemaphoreType.DMA,
        ],
    )
    return k(table, idx_pad)


def _post_body(x_ref, g_ref, dir_ref, mask_ref,
               wqT_ref, bq_ref, woT_ref, bo_ref,
               ln1g_ref, ln1b_ref, ln2g_ref, ln2b_ref,
               wf1T_ref, bf1_ref, wf2T_ref, bf2_ref,
               weC_ref, beC_ref, out_ref):
    inv = 1.0 / math.sqrt(HD)
    r = lax.broadcasted_iota(jnp.int32, (D, D), 0) // HD
    c2 = lax.broadcasted_iota(jnp.int32, (D, D), 1) // HD
    P = (r == c2).astype(jnp.float32)

    edgeC = dir_ref[...] * weC_ref[...] + beC_ref[...]
    maskC = mask_ref[...] > 0.5
    wqT = wqT_ref[...]

    for b in range(B):
        xb = x_ref[b]
        q = (jnp.dot(xb, wqT, preferred_element_type=jnp.float32)
             + bq_ref[...]) * inv
        qb = jnp.broadcast_to(q[:, None, :], (NB, DEG, D)).reshape(GB, D)
        kg = g_ref[:, b * D:(b + 1) * D]
        vg = g_ref[:, 2 * D + b * D:2 * D + (b + 1) * D]
        logitsC = jnp.dot(qb * kg, P, preferred_element_type=jnp.float32)
        logitsC = logitsC + edgeC
        logitsC = jnp.where(maskC, logitsC, -1e9)
        eC = jnp.exp(logitsC)
        numer = (eC * vg).reshape(NB, DEG, D).sum(axis=1)
        denom = eC.reshape(NB, DEG, D).sum(axis=1) + 1e-20
        agg = numer / denom

        h1 = xb + jnp.dot(agg, woT_ref[...],
                          preferred_element_type=jnp.float32) + bo_ref[...]
        m = jnp.mean(h1, axis=-1, keepdims=True)
        v = jnp.mean((h1 - m) ** 2, axis=-1, keepdims=True)
        h = (h1 - m) / jnp.sqrt(v + 1e-5) * ln1g_ref[...] + ln1b_ref[...]

        f = jnp.maximum(
            jnp.dot(h, wf1T_ref[...], preferred_element_type=jnp.float32)
            + bf1_ref[...], 0.0)
        f = jnp.dot(f, wf2T_ref[...],
                    preferred_element_type=jnp.float32) + bf2_ref[...]
        h2 = h + f
        m2 = jnp.mean(h2, axis=-1, keepdims=True)
        v2 = jnp.mean((h2 - m2) ** 2, axis=-1, keepdims=True)
        out_ref[b] = ((h2 - m2) / jnp.sqrt(v2 + 1e-5) * ln2g_ref[...]
                      + ln2b_ref[...])


def _post(x, g, dirE, maskE, WqT, bq, WoT, bo, ln1g, ln1b, ln2g, ln2b,
          Wf1T, bf1, Wf2T, bf2, weC, beC):
    full = lambda i: (0, 0)
    return pl.pallas_call(
        _post_body,
        grid=(NBLK,),
        in_specs=[
            pl.BlockSpec((B, NB, D), lambda i: (0, i, 0)),
            pl.BlockSpec((GB, TW), lambda i: (i, 0)),
            pl.BlockSpec((GB, 1), lambda i: (i, 0)),
            pl.BlockSpec((GB, 1), lambda i: (i, 0)),
            pl.BlockSpec((D, D), full),
            pl.BlockSpec((1, D), full),
            pl.BlockSpec((D, D), full),
            pl.BlockSpec((1, D), full),
            pl.BlockSpec((1, D), full),
            pl.BlockSpec((1, D), full),
            pl.BlockSpec((1, D), full),
            pl.BlockSpec((1, D), full),
            pl.BlockSpec((D, 2 * D), full),
            pl.BlockSpec((1, 2 * D), full),
            pl.BlockSpec((2 * D, D), full),
            pl.BlockSpec((1, D), full),
            pl.BlockSpec((1, D), full),
            pl.BlockSpec((1, D), full),
        ],
        out_specs=pl.BlockSpec((B, NB, D), lambda i: (0, i, 0)),
        out_shape=jax.ShapeDtypeStruct((B, N, D), jnp.float32),
    )(x, g, dirE, maskE, WqT, bq, WoT, bo, ln1g, ln1b, ln2g, ln2b,
      Wf1T, bf1, Wf2T, bf2, weC, beC)


def kernel(x, incoming_idx, incoming_dir, incoming_mask,
           Wq, bq, Wk, bk, Wv, bv, We, be, Wo, bo,
           ln1_g, ln1_b, ln2_g, ln2_b, Wf1, bf1, Wf2, bf2):
    table = _build_table(x, Wk.T, bk[None, :], Wv.T, bv[None, :])

    idx_flat = incoming_idx.reshape(-1)
    idx_pad = jnp.concatenate(
        [idx_flat, jnp.zeros((NPAD - NDEG,), jnp.int32)])
    g = _gather_rows(table, idx_pad)[:NDEG]

    dirE = incoming_dir.reshape(NDEG, 1)
    maskE = incoming_mask.reshape(NDEG, 1).astype(jnp.float32)
    weC = jnp.repeat(We[:, 0], HD)[None, :]
    beC = jnp.repeat(be, HD)[None, :]

    return _post(x, g, dirE, maskE, Wq.T, bq[None, :], Wo.T, bo[None, :],
                 ln1_g[None, :], ln1_b[None, :], ln2_g[None, :],
                 ln2_b[None, :], Wf1.T, bf1[None, :], Wf2.T, bf2[None, :],
                 weC, beC)


# speedup vs baseline: 1.4681x; 1.0872x over previous
import functools
import math

import jax
import jax.numpy as jnp
from jax import lax
from jax.experimental import pallas as pl
from jax.experimental.pallas import tpu as pltpu
from jax.experimental.pallas import tpu_sc as plsc

B, N, DEG, D, H = 2, 10000, 16, 128, 4
HD = D // H
NDEG = N * DEG
NW = 32
ROWS_PER_W = 5120
NPAD = NW * ROWS_PER_W
CH = 64
NCH = ROWS_PER_W // CH
TW = 4 * D

NB = 200
GB = NB * DEG
NBLK = N // NB

NBP = 2000
NPRE = N // NBP


def _pre_body(x_ref, wkT_ref, bk_ref, wvT_ref, bv_ref, out_ref):
    wkT = wkT_ref[...]
    wvT = wvT_ref[...]
    for b in range(B):
        xb = x_ref[b]
        out_ref[:, b * D:(b + 1) * D] = (
            jnp.dot(xb, wkT, preferred_element_type=jnp.float32) + bk_ref[...])
        out_ref[:, 2 * D + b * D:2 * D + (b + 1) * D] = (
            jnp.dot(xb, wvT, preferred_element_type=jnp.float32) + bv_ref[...])


def _build_table(x, WkT, bk, WvT, bv):
    return pl.pallas_call(
        _pre_body,
        grid=(NPRE,),
        in_specs=[
            pl.BlockSpec((B, NBP, D), lambda i: (0, i, 0)),
            pl.BlockSpec((D, D), lambda i: (0, 0)),
            pl.BlockSpec((1, D), lambda i: (0, 0)),
            pl.BlockSpec((D, D), lambda i: (0, 0)),
            pl.BlockSpec((1, D), lambda i: (0, 0)),
        ],
        out_specs=pl.BlockSpec((NBP, TW), lambda i: (i, 0)),
        out_shape=jax.ShapeDtypeStruct((N, TW), jnp.float32),
    )(x, WkT, bk, WvT, bv)


def _gather_body(table_hbm, idx_hbm, out_hbm, idx_v, rows_v, sem0, sem1):
    c = lax.axis_index("c")
    s = lax.axis_index("s")
    wid = s * 2 + c
    base = wid * ROWS_PER_W
    pltpu.sync_copy(idx_hbm.at[pl.ds(base, ROWS_PER_W)], idx_v)
    sems = (sem0, sem1)
    bufs = (rows_v.at[0], rows_v.at[1])

    def start_g(j, b):
        pltpu.async_copy(
            table_hbm.at[idx_v.at[pl.ds(j * CH, CH)]], bufs[b], sems[b])

    def finish(j, b):
        pltpu.make_async_copy(
            table_hbm.at[idx_v.at[pl.ds(0, CH)]], bufs[b], sems[b]).wait()
        pltpu.sync_copy(bufs[b], out_hbm.at[pl.ds(base + j * CH, CH)])

    start_g(0, 0)

    def body(p, carry):
        j0 = p * 2
        start_g(j0 + 1, 1)
        finish(j0, 0)

        @pl.when(p < NCH // 2 - 1)
        def _():
            start_g(j0 + 2, 0)

        finish(j0 + 1, 1)
        return carry

    lax.fori_loop(0, NCH // 2, body, 0)


def _gather_rows(table, idx_pad):
    mesh = plsc.VectorSubcoreMesh(core_axis_name="c", subcore_axis_name="s")
    k = pl.kernel(
        _gather_body,
        out_type=jax.ShapeDtypeStruct((NPAD, TW), jnp.float32),
        mesh=mesh,
        scratch_types=[
            pltpu.VMEM((ROWS_PER_W,), jnp.int32),
            pltpu.VMEM((2, CH, TW), jnp.float32),
            pltpu.SemaphoreType.DMA,
            pltpu.SemaphoreType.DMA,
        ],
    )
    return k(table, idx_pad)


def _post_body(x_ref, g_ref, dir_ref, mask_ref,
               wqT_ref, bq_ref, woT_ref, bo_ref,
               ln1g_ref, ln1b_ref, ln2g_ref, ln2b_ref,
               wf1T_ref, bf1_ref, wf2T_ref, bf2_ref,
               weC_ref, beC_ref, out_ref):
    inv = 1.0 / math.sqrt(HD)
    r = lax.broadcasted_iota(jnp.int32, (D, D), 0) // HD
    c2 = lax.broadcasted_iota(jnp.int32, (D, D), 1) // HD
    P = (r == c2).astype(jnp.float32)

    edgeC = dir_ref[...] * weC_ref[...] + beC_ref[...]
    maskC = mask_ref[...] > 0.5
    wqT = wqT_ref[...]

    for b in range(B):
        xb = x_ref[b]
        q = (jnp.dot(xb, wqT, preferred_element_type=jnp.float32)
             + bq_ref[...]) * inv
        qb = jnp.broadcast_to(q[:, None, :], (NB, DEG, D)).reshape(GB, D)
        kg = g_ref[:, b * D:(b + 1) * D]
        vg = g_ref[:, 2 * D + b * D:2 * D + (b + 1) * D]
        logitsC = jnp.dot(qb * kg, P, preferred_element_type=jnp.float32)
        logitsC = logitsC + edgeC
        logitsC = jnp.where(maskC, logitsC, -1e9)
        eC = jnp.exp(logitsC)
        numer = (eC * vg).reshape(NB, DEG, D).sum(axis=1)
        denom = eC.reshape(NB, DEG, D).sum(axis=1) + 1e-20
        agg = numer / denom

        h1 = xb + jnp.dot(agg, woT_ref[...],
                          preferred_element_type=jnp.float32) + bo_ref[...]
        m = jnp.mean(h1, axis=-1, keepdims=True)
        v = jnp.mean((h1 - m) ** 2, axis=-1, keepdims=True)
        h = (h1 - m) / jnp.sqrt(v + 1e-5) * ln1g_ref[...] + ln1b_ref[...]

        f = jnp.maximum(
            jnp.dot(h, wf1T_ref[...], preferred_element_type=jnp.float32)
            + bf1_ref[...], 0.0)
        f = jnp.dot(f, wf2T_ref[...],
                    preferred_element_type=jnp.float32) + bf2_ref[...]
        h2 = h + f
        m2 = jnp.mean(h2, axis=-1, keepdims=True)
        v2 = jnp.mean((h2 - m2) ** 2, axis=-1, keepdims=True)
        out_ref[b] = ((h2 - m2) / jnp.sqrt(v2 + 1e-5) * ln2g_ref[...]
                      + ln2b_ref[...])


def _post(x, g, dirE, maskE, WqT, bq, WoT, bo, ln1g, ln1b, ln2g, ln2b,
          Wf1T, bf1, Wf2T, bf2, weC, beC):
    full = lambda i: (0, 0)
    return pl.pallas_call(
        _post_body,
        grid=(NBLK,),
        in_specs=[
            pl.BlockSpec((B, NB, D), lambda i: (0, i, 0)),
            pl.BlockSpec((GB, TW), lambda i: (i, 0)),
            pl.BlockSpec((GB, 1), lambda i: (i, 0)),
            pl.BlockSpec((GB, 1), lambda i: (i, 0)),
            pl.BlockSpec((D, D), full),
            pl.BlockSpec((1, D), full),
            pl.BlockSpec((D, D), full),
            pl.BlockSpec((1, D), full),
            pl.BlockSpec((1, D), full),
            pl.BlockSpec((1, D), full),
            pl.BlockSpec((1, D), full),
            pl.BlockSpec((1, D), full),
            pl.BlockSpec((D, 2 * D), full),
            pl.BlockSpec((1, 2 * D), full),
            pl.BlockSpec((2 * D, D), full),
            pl.BlockSpec((1, D), full),
            pl.BlockSpec((1, D), full),
            pl.BlockSpec((1, D), full),
        ],
        out_specs=pl.BlockSpec((B, NB, D), lambda i: (0, i, 0)),
        out_shape=jax.ShapeDtypeStruct((B, N, D), jnp.float32),
    )(x, g, dirE, maskE, WqT, bq, WoT, bo, ln1g, ln1b, ln2g, ln2b,
      Wf1T, bf1, Wf2T, bf2, weC, beC)


def kernel(x, incoming_idx, incoming_dir, incoming_mask,
           Wq, bq, Wk, bk, Wv, bv, We, be, Wo, bo,
           ln1_g, ln1_b, ln2_g, ln2_b, Wf1, bf1, Wf2, bf2):
    table = _build_table(x, Wk.T, bk[None, :], Wv.T, bv[None, :])

    idx_flat = incoming_idx.reshape(-1)
    idx_pad = jnp.concatenate(
        [idx_flat, jnp.zeros((NPAD - NDEG,), jnp.int32)])
    g = _gather_rows(table, idx_pad)[:NDEG]

    dirE = incoming_dir.reshape(NDEG, 1)
    maskE = incoming_mask.reshape(NDEG, 1).astype(jnp.float32)
    weC = jnp.repeat(We[:, 0], HD)[None, :]
    beC = jnp.repeat(be, HD)[None, :]

    return _post(x, g, dirE, maskE, Wq.T, bq[None, :], Wo.T, bo[None, :],
                 ln1_g[None, :], ln1_b[None, :], ln2_g[None, :],
                 ln2_b[None, :], Wf1.T, bf1[None, :], Wf2.T, bf2[None, :],
                 weC, beC)


# speedup vs baseline: 2.1079x; 1.4358x over previous
import functools
import math

import jax
import jax.numpy as jnp
from jax import lax
from jax.experimental import pallas as pl
from jax.experimental.pallas import tpu as pltpu
from jax.experimental.pallas import tpu_sc as plsc

B, N, DEG, D, H = 2, 10000, 16, 128, 4
HD = D // H
NDEG = N * DEG
NW = 32
ROWS_PER_W = 5120
NPAD = NW * ROWS_PER_W
CH = 128
NCH = ROWS_PER_W // CH
TW = 2 * D

NB = 200
GB = NB * DEG
NBLK = N // NB

NBP = 2000
NPRE = N // NBP


def _pack_bf16(y):
    u = lax.bitcast_convert_type(y, jnp.uint32)
    r = (u + jnp.uint32(0x8000)) >> jnp.uint32(16)
    packed = r[:, :D // 2] | (r[:, D // 2:] << jnp.uint32(16))
    return lax.bitcast_convert_type(packed, jnp.int32)


def _unpack_bf16(gi):
    gu = lax.bitcast_convert_type(gi, jnp.uint32)
    lo = lax.bitcast_convert_type(gu << jnp.uint32(16), jnp.float32)
    hi = lax.bitcast_convert_type(gu & jnp.uint32(0xFFFF0000), jnp.float32)
    return jnp.concatenate([lo, hi], axis=1)


def _pre_body(x_ref, wkT_ref, bk_ref, wvT_ref, bv_ref, out_ref):
    wkT = wkT_ref[...]
    wvT = wvT_ref[...]
    hw = D // 2
    for b in range(B):
        xb = x_ref[b]
        out_ref[:, b * hw:(b + 1) * hw] = _pack_bf16(
            jnp.dot(xb, wkT, preferred_element_type=jnp.float32) + bk_ref[...])
        out_ref[:, D + b * hw:D + (b + 1) * hw] = _pack_bf16(
            jnp.dot(xb, wvT, preferred_element_type=jnp.float32) + bv_ref[...])


def _build_table(x, WkT, bk, WvT, bv):
    return pl.pallas_call(
        _pre_body,
        grid=(NPRE,),
        in_specs=[
            pl.BlockSpec((B, NBP, D), lambda i: (0, i, 0)),
            pl.BlockSpec((D, D), lambda i: (0, 0)),
            pl.BlockSpec((1, D), lambda i: (0, 0)),
            pl.BlockSpec((D, D), lambda i: (0, 0)),
            pl.BlockSpec((1, D), lambda i: (0, 0)),
        ],
        out_specs=pl.BlockSpec((NBP, TW), lambda i: (i, 0)),
        out_shape=jax.ShapeDtypeStruct((N, TW), jnp.int32),
    )(x, WkT, bk, WvT, bv)


def _gather_body(table_hbm, idx_hbm, out_hbm, idx_v, rows_v, sem0, sem1):
    c = lax.axis_index("c")
    s = lax.axis_index("s")
    wid = s * 2 + c
    base = wid * ROWS_PER_W
    pltpu.sync_copy(idx_hbm.at[pl.ds(base, ROWS_PER_W)], idx_v)
    sems = (sem0, sem1)
    bufs = (rows_v.at[0], rows_v.at[1])

    def start_g(j, b):
        pltpu.async_copy(
            table_hbm.at[idx_v.at[pl.ds(j * CH, CH)]], bufs[b], sems[b])

    def finish(j, b):
        pltpu.make_async_copy(
            table_hbm.at[idx_v.at[pl.ds(0, CH)]], bufs[b], sems[b]).wait()
        pltpu.sync_copy(bufs[b], out_hbm.at[pl.ds(base + j * CH, CH)])

    start_g(0, 0)

    def body(p, carry):
        j0 = p * 2
        start_g(j0 + 1, 1)
        finish(j0, 0)

        @pl.when(p < NCH // 2 - 1)
        def _():
            start_g(j0 + 2, 0)

        finish(j0 + 1, 1)
        return carry

    lax.fori_loop(0, NCH // 2, body, 0)


def _gather_rows(table, idx_pad):
    mesh = plsc.VectorSubcoreMesh(core_axis_name="c", subcore_axis_name="s")
    k = pl.kernel(
        _gather_body,
        out_type=jax.ShapeDtypeStruct((NPAD, TW), jnp.int32),
        mesh=mesh,
        scratch_types=[
            pltpu.VMEM((ROWS_PER_W,), jnp.int32),
            pltpu.VMEM((2, CH, TW), jnp.int32),
            pltpu.SemaphoreType.DMA,
            pltpu.SemaphoreType.DMA,
        ],
    )
    return k(table, idx_pad)


def _post_body(x_ref, g_ref, dir_ref, mask_ref,
               wqT_ref, bq_ref, woT_ref, bo_ref,
               ln1g_ref, ln1b_ref, ln2g_ref, ln2b_ref,
               wf1T_ref, bf1_ref, wf2T_ref, bf2_ref,
               weC_ref, beC_ref, out_ref):
    inv = 1.0 / math.sqrt(HD)
    r = lax.broadcasted_iota(jnp.int32, (D, D), 0) // HD
    c2 = lax.broadcasted_iota(jnp.int32, (D, D), 1) // HD
    P = (r == c2).astype(jnp.float32)

    edgeC = dir_ref[...] * weC_ref[...] + beC_ref[...]
    maskC = mask_ref[...] > 0.5
    wqT = wqT_ref[...]

    for b in range(B):
        xb = x_ref[b]
        q = (jnp.dot(xb, wqT, preferred_element_type=jnp.float32)
             + bq_ref[...]) * inv
        qb = jnp.broadcast_to(q[:, None, :], (NB, DEG, D)).reshape(GB, D)
        hw = D // 2
        kg = _unpack_bf16(g_ref[:, b * hw:(b + 1) * hw])
        vg = _unpack_bf16(g_ref[:, D + b * hw:D + (b + 1) * hw])
        logitsC = jnp.dot(qb * kg, P, preferred_element_type=jnp.float32)
        logitsC = logitsC + edgeC
        logitsC = jnp.where(maskC, logitsC, -1e9)
        eC = jnp.exp(logitsC)
        numer = (eC * vg).reshape(NB, DEG, D).sum(axis=1)
        denom = eC.reshape(NB, DEG, D).sum(axis=1) + 1e-20
        agg = numer / denom

        h1 = xb + jnp.dot(agg, woT_ref[...],
                          preferred_element_type=jnp.float32) + bo_ref[...]
        m = jnp.mean(h1, axis=-1, keepdims=True)
        v = jnp.mean((h1 - m) ** 2, axis=-1, keepdims=True)
        h = (h1 - m) / jnp.sqrt(v + 1e-5) * ln1g_ref[...] + ln1b_ref[...]

        f = jnp.maximum(
            jnp.dot(h, wf1T_ref[...], preferred_element_type=jnp.float32)
            + bf1_ref[...], 0.0)
        f = jnp.dot(f, wf2T_ref[...],
                    preferred_element_type=jnp.float32) + bf2_ref[...]
        h2 = h + f
        m2 = jnp.mean(h2, axis=-1, keepdims=True)
        v2 = jnp.mean((h2 - m2) ** 2, axis=-1, keepdims=True)
        out_ref[b] = ((h2 - m2) / jnp.sqrt(v2 + 1e-5) * ln2g_ref[...]
                      + ln2b_ref[...])


def _post(x, g, dirE, maskE, WqT, bq, WoT, bo, ln1g, ln1b, ln2g, ln2b,
          Wf1T, bf1, Wf2T, bf2, weC, beC):
    full = lambda i: (0, 0)
    return pl.pallas_call(
        _post_body,
        grid=(NBLK,),
        in_specs=[
            pl.BlockSpec((B, NB, D), lambda i: (0, i, 0)),
            pl.BlockSpec((GB, TW), lambda i: (i, 0)),
            pl.BlockSpec((GB, 1), lambda i: (i, 0)),
            pl.BlockSpec((GB, 1), lambda i: (i, 0)),
            pl.BlockSpec((D, D), full),
            pl.BlockSpec((1, D), full),
            pl.BlockSpec((D, D), full),
            pl.BlockSpec((1, D), full),
            pl.BlockSpec((1, D), full),
            pl.BlockSpec((1, D), full),
            pl.BlockSpec((1, D), full),
            pl.BlockSpec((1, D), full),
            pl.BlockSpec((D, 2 * D), full),
            pl.BlockSpec((1, 2 * D), full),
            pl.BlockSpec((2 * D, D), full),
            pl.BlockSpec((1, D), full),
            pl.BlockSpec((1, D), full),
            pl.BlockSpec((1, D), full),
        ],
        out_specs=pl.BlockSpec((B, NB, D), lambda i: (0, i, 0)),
        out_shape=jax.ShapeDtypeStruct((B, N, D), jnp.float32),
    )(x, g, dirE, maskE, WqT, bq, WoT, bo, ln1g, ln1b, ln2g, ln2b,
      Wf1T, bf1, Wf2T, bf2, weC, beC)


def kernel(x, incoming_idx, incoming_dir, incoming_mask,
           Wq, bq, Wk, bk, Wv, bv, We, be, Wo, bo,
           ln1_g, ln1_b, ln2_g, ln2_b, Wf1, bf1, Wf2, bf2):
    table = _build_table(x, Wk.T, bk[None, :], Wv.T, bv[None, :])

    idx_flat = incoming_idx.reshape(-1)
    idx_pad = jnp.concatenate(
        [idx_flat, jnp.zeros((NPAD - NDEG,), jnp.int32)])
    g = _gather_rows(table, idx_pad)

    dirE = incoming_dir.reshape(NDEG, 1)
    maskE = incoming_mask.reshape(NDEG, 1).astype(jnp.float32)
    weC = jnp.repeat(We[:, 0], HD)[None, :]
    beC = jnp.repeat(be, HD)[None, :]

    return _post(x, g, dirE, maskE, Wq.T, bq[None, :], Wo.T, bo[None, :],
                 ln1_g[None, :], ln1_b[None, :], ln2_g[None, :],
                 ln2_b[None, :], Wf1.T, bf1[None, :], Wf2.T, bf2[None, :],
                 weC, beC)
